# baseline (speedup 1.0000x reference)
import functools

import jax
import jax.numpy as jnp
from jax import lax
from jax.experimental import pallas as pl
from jax.experimental.pallas import tpu as pltpu
from jax.experimental.pallas import tpu_sc as plsc

D = 256
HID = 256
N_NODES = 10000
N_EDGES = 160000

NC = 2
NS = 16
NW = NC * NS

PACK = HID // 2
CHUNK = 128
E_SPLIT = (39680, 39680, 39680, 40960)
SEG_SPLIT = (40960, 40960, 40960, 40960)
NBUF = 2
LOOK = 1

EDGE_BLK = 1280
NODE_BLK = 1000


def _bf16_bits_hi(x):
    r = x.astype(jnp.bfloat16).astype(jnp.float32)
    return lax.bitcast_convert_type(r, jnp.uint32)


def _proj_body(n_ref, w_ref, out_ref):
    h = jnp.dot(n_ref[...], w_ref[0], preferred_element_type=jnp.float32)
    lo = jnp.right_shift(_bf16_bits_hi(h[:, :PACK]), jnp.uint32(16))
    hi = jnp.bitwise_and(_bf16_bits_hi(h[:, PACK:]), jnp.uint32(0xFFFF0000))
    out_ref[...] = lax.bitcast_convert_type(jnp.bitwise_or(lo, hi),
                                            jnp.float32)


def _node_projections(nfeat_bf, w1_sd_bf):
    grid = (2, N_NODES // NODE_BLK)
    return pl.pallas_call(
        _proj_body,
        grid=grid,
        in_specs=[
            pl.BlockSpec((NODE_BLK, D), lambda s, j: (j, 0)),
            pl.BlockSpec((1, D, HID), lambda s, j: (s, 0, 0)),
        ],
        out_specs=pl.BlockSpec((NODE_BLK, PACK),
                               lambda s, j: (s * (N_NODES // NODE_BLK) + j, 0)),
        out_shape=jax.ShapeDtypeStruct((2 * N_NODES, PACK), jnp.float32),
    )(nfeat_bf, w1_sd_bf)


@functools.cache
def _make_sc_gather(cpw):
    total = cpw * CHUNK * NW

    @functools.partial(
        pl.kernel,
        mesh=plsc.VectorSubcoreMesh(core_axis_name="c", subcore_axis_name="s"),
        out_type=jax.ShapeDtypeStruct((total, PACK), jnp.float32),
        scratch_types=[
            pltpu.VMEM((cpw * CHUNK,), jnp.int32),
            pltpu.VMEM_SHARED((N_NODES, PACK), jnp.float32),
        ] + [pltpu.VMEM((CHUNK, PACK), jnp.float32) for _ in range(NBUF)]
          + [pltpu.SemaphoreType.DMA for _ in range(2 * NBUF)],
    )
    def _sc_gather(table_hbm, idx_hbm, out_hbm, idx_v, tbl_sp,
                   *bufs_and_sems):
        bufs = bufs_and_sems[:NBUF]
        gsems = bufs_and_sems[NBUF:2 * NBUF]
        ssems = bufs_and_sems[2 * NBUF:]
        cc = lax.axis_index("c")
        sid = lax.axis_index("s")
        wid = cc * NS + sid
        base = wid * cpw

        half = N_NODES // 2
        tbl_base = pl.multiple_of(cc * N_NODES, 8)
        for k in range(2):
            @pl.when(sid == k)
            def _():
                pltpu.sync_copy(
                    table_hbm.at[pl.ds(tbl_base + k * half, half)],
                    tbl_sp.at[pl.ds(k * half, half)])
        pltpu.sync_copy(idx_hbm.at[pl.ds(base * CHUNK, cpw * CHUNK)], idx_v)
        plsc.subcore_barrier()

        def out_at(c):
            return out_hbm.at[pl.ds((base + c) * CHUNK, CHUNK)]

        def idx_at(c):
            return idx_v.at[pl.ds(pl.multiple_of(c * CHUNK, CHUNK), CHUNK)]

        for k in range(LOOK):
            pltpu.async_copy(tbl_sp.at[idx_at(k)], bufs[k], gsems[k])

        def body(jo, carry):
            for b in range(NBUF):
                c = jo * NBUF + b
                bn = (b + LOOK) % NBUF
                cn = jnp.minimum(c + LOOK, cpw - 1)
                cp = jnp.maximum(c - (NBUF - LOOK), 0)

                @pl.when(c >= NBUF - LOOK)
                def _():
                    pltpu.make_async_copy(bufs[bn], out_at(cp),
                                          ssems[bn]).wait()

                @pl.when(c + LOOK < cpw)
                def _():
                    pltpu.async_copy(tbl_sp.at[idx_at(cn)], bufs[bn],
                                     gsems[bn])

                pltpu.make_async_copy(tbl_sp.at[idx_at(c)], bufs[b],
                                      gsems[b]).wait()
                pltpu.async_copy(bufs[b], out_at(c), ssems[b])
            return carry

        lax.fori_loop(0, cpw // NBUF, body, 0)

        for k in range(NBUF - LOOK):
            c = cpw - (NBUF - LOOK) + k
            pltpu.make_async_copy(bufs[c % NBUF], out_at(c),
                                  ssems[c % NBUF]).wait()

    return _sc_gather


def _unpack_halves(words_f32):
    w = lax.bitcast_convert_type(words_f32, jnp.uint32)
    lo = lax.bitcast_convert_type(
        jnp.left_shift(w, jnp.uint32(16)), jnp.float32)
    hi = lax.bitcast_convert_type(
        jnp.bitwise_and(w, jnp.uint32(0xFFFF0000)), jnp.float32)
    return lo, hi


def _edge_body(e_ref, gs_ref, gd_ref, w1e_ref, w2a_ref, w2b_ref, b1_ref,
               b2_ref, sc_ref, bi_ref, out_ref):
    x = e_ref[...]
    gsa, gsb = _unpack_halves(gs_ref[...])
    gda, gdb = _unpack_halves(gd_ref[...])
    h1 = jnp.dot(x.astype(jnp.bfloat16), w1e_ref[...],
                 preferred_element_type=jnp.float32)
    h1a = h1[:, :PACK] + gsa + gda + b1_ref[:, :PACK]
    h1b = h1[:, PACK:] + gsb + gdb + b1_ref[:, PACK:]
    h1a = h1a * jax.nn.sigmoid(h1a)
    h1b = h1b * jax.nn.sigmoid(h1b)
    h = jnp.dot(h1a.astype(jnp.bfloat16), w2a_ref[...],
                preferred_element_type=jnp.float32)
    h = h + jnp.dot(h1b.astype(jnp.bfloat16), w2b_ref[...],
                    preferred_element_type=jnp.float32)
    h = h + b2_ref[...]
    mean = jnp.mean(h, axis=-1, keepdims=True)
    c = h - mean
    var = jnp.mean(c * c, axis=-1, keepdims=True)
    out_ref[...] = c * lax.rsqrt(var + 1e-5) * sc_ref[...] + bi_ref[...] + x


def _edge_body_alias(prev_ref, *rest):
    _edge_body(*rest)


def _edge_mlp(efeat, gathered, w1e_bf, w2a_bf, w2b_bf, b1, b2,
              ln_scale, ln_bias, n_edges, blk_off, seg, prev=None):
    grid = (n_edges // EDGE_BLK,)
    full = lambda i: (0, 0)
    in_specs = [
        pl.BlockSpec((EDGE_BLK, D), lambda i: (i + blk_off, 0)),
        pl.BlockSpec((EDGE_BLK, PACK), lambda i: (i, 0)),
        pl.BlockSpec((EDGE_BLK, PACK), lambda i: (i + seg // EDGE_BLK, 0)),
        pl.BlockSpec((D, HID), full),
        pl.BlockSpec((PACK, D), full),
        pl.BlockSpec((PACK, D), full),
        pl.BlockSpec((1, HID), full),
        pl.BlockSpec((1, D), full),
        pl.BlockSpec((1, D), full),
        pl.BlockSpec((1, D), full),
    ]
    args = (efeat, gathered, gathered, w1e_bf, w2a_bf, w2b_bf, b1, b2,
            ln_scale, ln_bias)
    body = _edge_body
    aliases = {}
    if prev is not None:
        in_specs = [pl.BlockSpec((8, D), full)] + in_specs
        args = (prev,) + args
        body = _edge_body_alias
        aliases = {0: 0}
    return pl.pallas_call(
        body,
        grid=grid,
        in_specs=in_specs,
        out_specs=pl.BlockSpec((EDGE_BLK, D), lambda i: (i + blk_off, 0)),
        out_shape=jax.ShapeDtypeStruct((N_EDGES, D), jnp.float32),
        input_output_aliases=aliases,
    )(*args)


def kernel(efeat, nfeat, edge_index, w1, b1, w2, b2, ln_scale, ln_bias):
    src = edge_index[0].astype(jnp.int32)
    dst = edge_index[1].astype(jnp.int32)

    w1e_bf = w1[:D].astype(jnp.bfloat16)
    w2a_bf = w2[:PACK].astype(jnp.bfloat16)
    w2b_bf = w2[PACK:].astype(jnp.bfloat16)
    w1_sd = jnp.stack([w1[D:2 * D], w1[2 * D:]])

    packed = _node_projections(nfeat, w1_sd)

    b1r = b1.reshape(1, HID)
    b2r = b2.reshape(1, D)
    scr = ln_scale.reshape(1, D)
    bir = ln_bias.reshape(1, D)

    gathers = []
    e0 = 0
    for n_e, seg in zip(E_SPLIT, SEG_SPLIT):
        s_sl = src[e0:e0 + n_e]
        d_sl = dst[e0:e0 + n_e]
        if n_e == seg:
            idx = jnp.concatenate([s_sl, d_sl])
        else:
            idx = jnp.zeros((2 * seg,), jnp.int32)
            idx = idx.at[:n_e].set(s_sl)
            idx = idx.at[seg:seg + n_e].set(d_sl)
        gathers.append(_make_sc_gather(2 * seg // CHUNK // NW)(packed, idx))
        e0 += n_e

    out = None
    e0 = 0
    for (n_e, seg), gathered in zip(zip(E_SPLIT, SEG_SPLIT), gathers):
        out = _edge_mlp(efeat, gathered, w1e_bf, w2a_bf, w2b_bf,
                        b1r, b2r, scr, bir, n_e, e0 // EDGE_BLK, seg,
                        prev=out)
        e0 += n_e
    return (out, nfeat)


# speedup vs baseline: 1.0176x; 1.0176x over previous
import functools

import jax
import jax.numpy as jnp
from jax import lax
from jax.experimental import pallas as pl
from jax.experimental.pallas import tpu as pltpu
from jax.experimental.pallas import tpu_sc as plsc

D = 256
HID = 256
N_NODES = 10000
N_EDGES = 160000

NC = 2
NS = 16
NW = NC * NS

PACK = HID // 2
CHUNK = 128
E_SPLIT = (20480, 61440, 78080)
SEG_SPLIT = (20480, 61440, 81920)
NBUF = 2
LOOK = 1

EDGE_BLK = 1280
NODE_BLK = 1000


def _bf16_bits_hi(x):
    r = x.astype(jnp.bfloat16).astype(jnp.float32)
    return lax.bitcast_convert_type(r, jnp.uint32)


def _proj_body(n_ref, w_ref, out_ref):
    h = jnp.dot(n_ref[...], w_ref[0], preferred_element_type=jnp.float32)
    lo = jnp.right_shift(_bf16_bits_hi(h[:, :PACK]), jnp.uint32(16))
    hi = jnp.bitwise_and(_bf16_bits_hi(h[:, PACK:]), jnp.uint32(0xFFFF0000))
    out_ref[...] = lax.bitcast_convert_type(jnp.bitwise_or(lo, hi),
                                            jnp.float32)


def _node_projections(nfeat_bf, w1_sd_bf):
    grid = (2, N_NODES // NODE_BLK)
    return pl.pallas_call(
        _proj_body,
        grid=grid,
        in_specs=[
            pl.BlockSpec((NODE_BLK, D), lambda s, j: (j, 0)),
            pl.BlockSpec((1, D, HID), lambda s, j: (s, 0, 0)),
        ],
        out_specs=pl.BlockSpec((NODE_BLK, PACK),
                               lambda s, j: (s * (N_NODES // NODE_BLK) + j, 0)),
        out_shape=jax.ShapeDtypeStruct((2 * N_NODES, PACK), jnp.float32),
    )(nfeat_bf, w1_sd_bf)


@functools.cache
def _make_sc_gather(cpw):
    total = cpw * CHUNK * NW

    @functools.partial(
        pl.kernel,
        mesh=plsc.VectorSubcoreMesh(core_axis_name="c", subcore_axis_name="s"),
        out_type=jax.ShapeDtypeStruct((total, PACK), jnp.float32),
        scratch_types=[
            pltpu.VMEM((cpw * CHUNK,), jnp.int32),
            pltpu.VMEM_SHARED((N_NODES, PACK), jnp.float32),
        ] + [pltpu.VMEM((CHUNK, PACK), jnp.float32) for _ in range(NBUF)]
          + [pltpu.SemaphoreType.DMA for _ in range(2 * NBUF)],
    )
    def _sc_gather(table_hbm, idx_hbm, out_hbm, idx_v, tbl_sp,
                   *bufs_and_sems):
        bufs = bufs_and_sems[:NBUF]
        gsems = bufs_and_sems[NBUF:2 * NBUF]
        ssems = bufs_and_sems[2 * NBUF:]
        cc = lax.axis_index("c")
        sid = lax.axis_index("s")
        wid = cc * NS + sid
        base = wid * cpw

        half = N_NODES // 2
        tbl_base = pl.multiple_of(cc * N_NODES, 8)
        for k in range(2):
            @pl.when(sid == k)
            def _():
                pltpu.sync_copy(
                    table_hbm.at[pl.ds(tbl_base + k * half, half)],
                    tbl_sp.at[pl.ds(k * half, half)])
        pltpu.sync_copy(idx_hbm.at[pl.ds(base * CHUNK, cpw * CHUNK)], idx_v)
        plsc.subcore_barrier()

        def out_at(c):
            return out_hbm.at[pl.ds((base + c) * CHUNK, CHUNK)]

        def idx_at(c):
            return idx_v.at[pl.ds(pl.multiple_of(c * CHUNK, CHUNK), CHUNK)]

        for k in range(LOOK):
            pltpu.async_copy(tbl_sp.at[idx_at(k)], bufs[k], gsems[k])

        def body(jo, carry):
            for b in range(NBUF):
                c = jo * NBUF + b
                bn = (b + LOOK) % NBUF
                cn = jnp.minimum(c + LOOK, cpw - 1)
                cp = jnp.maximum(c - (NBUF - LOOK), 0)

                @pl.when(c >= NBUF - LOOK)
                def _():
                    pltpu.make_async_copy(bufs[bn], out_at(cp),
                                          ssems[bn]).wait()

                @pl.when(c + LOOK < cpw)
                def _():
                    pltpu.async_copy(tbl_sp.at[idx_at(cn)], bufs[bn],
                                     gsems[bn])

                pltpu.make_async_copy(tbl_sp.at[idx_at(c)], bufs[b],
                                      gsems[b]).wait()
                pltpu.async_copy(bufs[b], out_at(c), ssems[b])
            return carry

        lax.fori_loop(0, cpw // NBUF, body, 0)

        for k in range(NBUF - LOOK):
            c = cpw - (NBUF - LOOK) + k
            pltpu.make_async_copy(bufs[c % NBUF], out_at(c),
                                  ssems[c % NBUF]).wait()

    return _sc_gather


def _unpack_halves(words_f32):
    w = lax.bitcast_convert_type(words_f32, jnp.uint32)
    lo = lax.bitcast_convert_type(
        jnp.left_shift(w, jnp.uint32(16)), jnp.float32)
    hi = lax.bitcast_convert_type(
        jnp.bitwise_and(w, jnp.uint32(0xFFFF0000)), jnp.float32)
    return lo, hi


def _edge_body(e_ref, gs_ref, gd_ref, w1e_ref, w2a_ref, w2b_ref, b1_ref,
               b2_ref, sc_ref, bi_ref, out_ref):
    x = e_ref[...]
    gsa, gsb = _unpack_halves(gs_ref[...])
    gda, gdb = _unpack_halves(gd_ref[...])
    h1 = jnp.dot(x.astype(jnp.bfloat16), w1e_ref[...],
                 preferred_element_type=jnp.float32)
    h1a = h1[:, :PACK] + gsa + gda + b1_ref[:, :PACK]
    h1b = h1[:, PACK:] + gsb + gdb + b1_ref[:, PACK:]
    h1a = h1a * jax.nn.sigmoid(h1a)
    h1b = h1b * jax.nn.sigmoid(h1b)
    h = jnp.dot(h1a.astype(jnp.bfloat16), w2a_ref[...],
                preferred_element_type=jnp.float32)
    h = h + jnp.dot(h1b.astype(jnp.bfloat16), w2b_ref[...],
                    preferred_element_type=jnp.float32)
    h = h + b2_ref[...]
    mean = jnp.mean(h, axis=-1, keepdims=True)
    c = h - mean
    var = jnp.mean(c * c, axis=-1, keepdims=True)
    out_ref[...] = c * lax.rsqrt(var + 1e-5) * sc_ref[...] + bi_ref[...] + x


def _edge_body_alias(prev_ref, *rest):
    _edge_body(*rest)


def _edge_mlp(efeat, gathered, w1e_bf, w2a_bf, w2b_bf, b1, b2,
              ln_scale, ln_bias, n_edges, blk_off, seg, prev=None):
    grid = (n_edges // EDGE_BLK,)
    full = lambda i: (0, 0)
    in_specs = [
        pl.BlockSpec((EDGE_BLK, D), lambda i: (i + blk_off, 0)),
        pl.BlockSpec((EDGE_BLK, PACK), lambda i: (i, 0)),
        pl.BlockSpec((EDGE_BLK, PACK), lambda i: (i + seg // EDGE_BLK, 0)),
        pl.BlockSpec((D, HID), full),
        pl.BlockSpec((PACK, D), full),
        pl.BlockSpec((PACK, D), full),
        pl.BlockSpec((1, HID), full),
        pl.BlockSpec((1, D), full),
        pl.BlockSpec((1, D), full),
        pl.BlockSpec((1, D), full),
    ]
    args = (efeat, gathered, gathered, w1e_bf, w2a_bf, w2b_bf, b1, b2,
            ln_scale, ln_bias)
    body = _edge_body
    aliases = {}
    if prev is not None:
        in_specs = [pl.BlockSpec((8, D), full)] + in_specs
        args = (prev,) + args
        body = _edge_body_alias
        aliases = {0: 0}
    return pl.pallas_call(
        body,
        grid=grid,
        in_specs=in_specs,
        out_specs=pl.BlockSpec((EDGE_BLK, D), lambda i: (i + blk_off, 0)),
        out_shape=jax.ShapeDtypeStruct((N_EDGES, D), jnp.float32),
        input_output_aliases=aliases,
    )(*args)


def kernel(efeat, nfeat, edge_index, w1, b1, w2, b2, ln_scale, ln_bias):
    src = edge_index[0].astype(jnp.int32)
    dst = edge_index[1].astype(jnp.int32)

    w1e_bf = w1[:D].astype(jnp.bfloat16)
    w2a_bf = w2[:PACK].astype(jnp.bfloat16)
    w2b_bf = w2[PACK:].astype(jnp.bfloat16)
    w1_sd = jnp.stack([w1[D:2 * D], w1[2 * D:]])

    packed = _node_projections(nfeat, w1_sd)

    b1r = b1.reshape(1, HID)
    b2r = b2.reshape(1, D)
    scr = ln_scale.reshape(1, D)
    bir = ln_bias.reshape(1, D)

    gathers = []
    e0 = 0
    for n_e, seg in zip(E_SPLIT, SEG_SPLIT):
        s_sl = src[e0:e0 + n_e]
        d_sl = dst[e0:e0 + n_e]
        if n_e == seg:
            idx = jnp.concatenate([s_sl, d_sl])
        else:
            idx = jnp.zeros((2 * seg,), jnp.int32)
            idx = idx.at[:n_e].set(s_sl)
            idx = idx.at[seg:seg + n_e].set(d_sl)
        gathers.append(_make_sc_gather(2 * seg // CHUNK // NW)(packed, idx))
        e0 += n_e

    out = None
    e0 = 0
    for (n_e, seg), gathered in zip(zip(E_SPLIT, SEG_SPLIT), gathers):
        out = _edge_mlp(efeat, gathered, w1e_bf, w2a_bf, w2b_bf,
                        b1r, b2r, scr, bir, n_e, e0 // EDGE_BLK, seg,
                        prev=out)
        e0 += n_e
    return (out, nfeat)
